# baseline (speedup 1.0000x reference)
import functools

import jax
import jax.numpy as jnp
from jax import lax
from jax.experimental import pallas as pl
from jax.experimental.pallas import tpu as pltpu
from jax.experimental.pallas import tpu_sc as plsc

E = 64
D_MODEL = 768
D_FF = 2048
T = 2048

TILE = 16
NT = 192
PAD = NT * TILE

_CHUNK_T = 128
_CHUNK_P = 512


def _meta_compute(ids, pos_ref, src_ref, ps_ref, nt_ref):
    e_iota = lax.broadcasted_iota(jnp.int32, (T, E), 1)
    onehot = (ids == e_iota).astype(jnp.float32)

    r_iota = lax.broadcasted_iota(jnp.int32, (_CHUNK_T, _CHUNK_T), 0)
    c_iota = lax.broadcasted_iota(jnp.int32, (_CHUNK_T, _CHUNK_T), 1)
    tri = (c_iota <= r_iota).astype(jnp.float32)
    nchunk = T // _CHUNK_T
    local = [
        jnp.dot(tri, onehot[k * _CHUNK_T:(k + 1) * _CHUNK_T, :],
                preferred_element_type=jnp.float32)
        for k in range(nchunk)
    ]
    offset = jnp.zeros((1, E), jnp.float32)
    chunks = []
    for k in range(nchunk):
        chunks.append(local[k] + offset)
        offset = offset + local[k][_CHUNK_T - 1:_CHUNK_T, :]
    csum = jnp.concatenate(chunks, axis=0)

    counts = csum[T - 1:T, :]
    tiles = jnp.floor((counts + (TILE - 1)) * (1.0 / TILE))
    e_sq_r = lax.broadcasted_iota(jnp.int32, (E, E), 0)
    e_sq_c = lax.broadcasted_iota(jnp.int32, (E, E), 1)
    stri = (e_sq_r < e_sq_c).astype(jnp.float32)
    tile_start = jnp.dot(tiles, stri, preferred_element_type=jnp.float32)
    pad_start = tile_start * float(TILE)

    rank = jnp.sum(onehot * (csum - 1.0), axis=1, keepdims=True)
    pos_f = jnp.sum(onehot * pad_start, axis=1, keepdims=True) + rank
    pos_ref[...] = pos_f.astype(jnp.int32)

    ps_ref[...] = pad_start.astype(jnp.int32)
    nt_ref[...] = tiles.astype(jnp.int32)

    pos_i = pos_f.astype(jnp.int32)
    tcol = lax.broadcasted_iota(jnp.int32, (T, 1), 0).astype(jnp.float32) + 1.0
    for r in range(PAD // _CHUNK_P):
        p_iota = lax.broadcasted_iota(jnp.int32, (T, _CHUNK_P), 1) + r * _CHUNK_P
        hit = (pos_i == p_iota).astype(jnp.float32)
        srcv = jnp.sum(hit * tcol, axis=0, keepdims=True)
        prow = (lax.broadcasted_iota(jnp.int32, (1, _CHUNK_P), 1)
                + (r * _CHUNK_P) % T).astype(jnp.float32)
        src_ref[r:r + 1, :] = jnp.where(srcv > 0.0, srcv - 1.0, prow).astype(jnp.int32)


def _route_meta2_body(x_ref, ra_ref, rb_ref,
                      pa_ref, sa_ref, psa_ref, nta_ref,
                      pb_ref, sb_ref, psb_ref, ntb_ref):
    x = x_ref[...]
    la = jnp.dot(x, ra_ref[...], preferred_element_type=jnp.float32)
    lb = jnp.dot(x, rb_ref[...], preferred_element_type=jnp.float32)
    ids_a = jnp.argmax(la, axis=1, keepdims=True).astype(jnp.int32)
    ids_b = jnp.argmax(lb, axis=1, keepdims=True).astype(jnp.int32)
    _meta_compute(ids_a, pa_ref, sa_ref, psa_ref, nta_ref)
    _meta_compute(ids_b, pb_ref, sb_ref, psb_ref, ntb_ref)


def _route_meta2(x, rwa, rwb):
    d = x.shape[1]
    meta_shapes = [
        jax.ShapeDtypeStruct((T, 1), jnp.int32),
        jax.ShapeDtypeStruct((PAD // _CHUNK_P, _CHUNK_P), jnp.int32),
        jax.ShapeDtypeStruct((1, E), jnp.int32),
        jax.ShapeDtypeStruct((1, E), jnp.int32),
    ]
    meta_specs = [
        pl.BlockSpec((T, 1), lambda: (0, 0)),
        pl.BlockSpec((PAD // _CHUNK_P, _CHUNK_P), lambda: (0, 0)),
        pl.BlockSpec((1, E), lambda: (0, 0)),
        pl.BlockSpec((1, E), lambda: (0, 0)),
    ]
    return pl.pallas_call(
        _route_meta2_body,
        out_shape=meta_shapes + meta_shapes,
        in_specs=[
            pl.BlockSpec((T, d), lambda: (0, 0)),
            pl.BlockSpec((d, E), lambda: (0, 0)),
            pl.BlockSpec((d, E), lambda: (0, 0)),
        ],
        out_specs=meta_specs + meta_specs,
    )(x, rwa, rwb)


def _gather_loop(info, n, d, table_hbm, idx_hbm, out_hbm, idx_v, rows_v, sem):
    nw = info.num_cores * info.num_subcores
    b_per_w = n // nw
    chunk = idx_v.shape[0]
    wid = lax.axis_index("s") * info.num_cores + lax.axis_index("c")
    for ci in range(b_per_w // chunk):
        base = wid * b_per_w + ci * chunk
        pltpu.sync_copy(idx_hbm.at[pl.ds(base, chunk)], idx_v)
        pltpu.async_copy(table_hbm.at[idx_v], rows_v, sem).wait()
        pltpu.sync_copy(rows_v, out_hbm.at[pl.ds(base, chunk)])


def _chunk_rows(n, d, info):
    b_per_w = n // (info.num_cores * info.num_subcores)
    budget_rows = (192 * 1024) // (d * 4)
    chunk = b_per_w
    while chunk > 8 and (chunk > budget_rows or chunk % 8 != 0):
        chunk //= 2
    return chunk


def _sc_gather(table, idx):
    n, d = idx.shape[0], table.shape[1]
    info = plsc.get_sparse_core_info()
    chunk = _chunk_rows(n, d, info)
    mesh = plsc.VectorSubcoreMesh(core_axis_name="c", subcore_axis_name="s")

    @functools.partial(
        pl.kernel,
        mesh=mesh,
        out_type=jax.ShapeDtypeStruct((n, d), jnp.float32),
        scratch_types=[
            pltpu.VMEM((chunk,), jnp.int32),
            pltpu.VMEM((chunk, d), jnp.float32),
            pltpu.SemaphoreType.DMA,
        ],
    )
    def k(table_hbm, idx_hbm, out_hbm, idx_v, rows_v, sem):
        _gather_loop(info, n, d, table_hbm, idx_hbm, out_hbm, idx_v, rows_v, sem)

    return k(table, idx)


def _sc_gather2(table_a, idx_a, table_b, idx_b):
    na, da = idx_a.shape[0], table_a.shape[1]
    nb, db = idx_b.shape[0], table_b.shape[1]
    info = plsc.get_sparse_core_info()
    ca = _chunk_rows(na, da, info)
    cb = _chunk_rows(nb, db, info)
    mesh = plsc.VectorSubcoreMesh(core_axis_name="c", subcore_axis_name="s")

    @functools.partial(
        pl.kernel,
        mesh=mesh,
        out_type=[
            jax.ShapeDtypeStruct((na, da), jnp.float32),
            jax.ShapeDtypeStruct((nb, db), jnp.float32),
        ],
        scratch_types=[
            pltpu.VMEM((ca,), jnp.int32),
            pltpu.VMEM((ca, da), jnp.float32),
            pltpu.VMEM((cb,), jnp.int32),
            pltpu.VMEM((cb, db), jnp.float32),
            pltpu.SemaphoreType.DMA,
        ],
    )
    def k(ta_hbm, ia_hbm, tb_hbm, ib_hbm, oa_hbm, ob_hbm,
          ia_v, ra_v, ib_v, rb_v, sem):
        _gather_loop(info, na, da, ta_hbm, ia_hbm, oa_hbm, ia_v, ra_v, sem)
        _gather_loop(info, nb, db, tb_hbm, ib_hbm, ob_hbm, ib_v, rb_v, sem)

    return k(table_a, idx_a, table_b, idx_b)


def _grouped_mm_body(ps_ref, nt_ref, x_ref, whi_ref, wlo_ref, o_ref):
    e = pl.program_id(1)
    start = ps_ref[e]
    whi = whi_ref[0]
    wlo = wlo_ref[0]
    hk = whi.shape[0]

    def tile_body(k, _):
        off = pl.multiple_of(start + k * TILE, TILE)
        xr = x_ref[pl.ds(off, TILE), :]
        o_ref[pl.ds(off, TILE), :] = (
            jnp.dot(xr[:, :hk], whi, preferred_element_type=jnp.float32)
            + jnp.dot(xr[:, hk:], wlo, preferred_element_type=jnp.float32))
        return _

    lax.fori_loop(0, nt_ref[e], tile_body, None)


def _grouped_mm(xs, w, ps, nt, nsplit):
    d_in, d_out = w.shape[1], w.shape[2]
    half = d_out // nsplit
    spec = pltpu.PrefetchScalarGridSpec(
        num_scalar_prefetch=2,
        grid=(nsplit, E),
        in_specs=[
            pl.BlockSpec((PAD, d_in), lambda j, e, ps, nt: (0, 0)),
            pl.BlockSpec((1, d_in // 2, half), lambda j, e, ps, nt: (e, 0, j)),
            pl.BlockSpec((1, d_in // 2, half), lambda j, e, ps, nt: (e, 1, j)),
        ],
        out_specs=pl.BlockSpec((PAD, half), lambda j, e, ps, nt: (0, j)),
    )
    return pl.pallas_call(
        _grouped_mm_body,
        grid_spec=spec,
        out_shape=jax.ShapeDtypeStruct((PAD, d_out), jnp.float32),
    )(ps, nt, xs, w, w)


_CBLK = 256


def _combine_meta_body(g_ref, u_ref, rd_ref, h_ref, pc_ref, sc_ref,
                       psc_ref, ntc_ref, ids_acc):
    i = pl.program_id(0)
    g = g_ref[...]
    h = g * lax.logistic(g) * u_ref[...]
    h_ref[...] = h
    lc = jnp.dot(h, rd_ref[...], preferred_element_type=jnp.float32)
    ids_acc[pl.ds(i * _CBLK, _CBLK), :] = (
        jnp.argmax(lc, axis=1, keepdims=True).astype(jnp.int32))

    @pl.when(i == T // _CBLK - 1)
    def _finish():
        _meta_compute(ids_acc[...], pc_ref, sc_ref, psc_ref, ntc_ref)


def _combine_meta(g, u, rdown):
    return pl.pallas_call(
        _combine_meta_body,
        grid=(T // _CBLK,),
        out_shape=[
            jax.ShapeDtypeStruct((T, D_FF), jnp.float32),
            jax.ShapeDtypeStruct((T, 1), jnp.int32),
            jax.ShapeDtypeStruct((PAD // _CHUNK_P, _CHUNK_P), jnp.int32),
            jax.ShapeDtypeStruct((1, E), jnp.int32),
            jax.ShapeDtypeStruct((1, E), jnp.int32),
        ],
        in_specs=[
            pl.BlockSpec((_CBLK, D_FF), lambda i: (i, 0)),
            pl.BlockSpec((_CBLK, D_FF), lambda i: (i, 0)),
            pl.BlockSpec((D_FF, E), lambda i: (0, 0)),
        ],
        out_specs=[
            pl.BlockSpec((_CBLK, D_FF), lambda i: (i, 0)),
            pl.BlockSpec((T, 1), lambda i: (0, 0)),
            pl.BlockSpec((PAD // _CHUNK_P, _CHUNK_P), lambda i: (0, 0)),
            pl.BlockSpec((1, E), lambda i: (0, 0)),
            pl.BlockSpec((1, E), lambda i: (0, 0)),
        ],
        scratch_shapes=[pltpu.VMEM((T, 1), jnp.int32)],
    )(g, u, rdown)


def kernel(hidden_state, router_gate, w_gate, router_up, w_up, router_down, w_down):
    b, s, d = hidden_state.shape
    x = hidden_state.reshape(b * s, d)

    (pos_a, src_a, ps_a, nt_a,
     pos_b, src_b, ps_b, nt_b) = _route_meta2(x, router_gate, router_up)

    xs_a = _sc_gather(x, src_a.reshape(PAD))
    ys_a = _grouped_mm(xs_a, w_gate, ps_a.reshape(E), nt_a.reshape(E), 2)
    xs_b = _sc_gather(x, src_b.reshape(PAD))
    g = _sc_gather(ys_a, pos_a.reshape(T))
    ys_b = _grouped_mm(xs_b, w_up, ps_b.reshape(E), nt_b.reshape(E), 2)
    u = _sc_gather(ys_b, pos_b.reshape(T))

    h, pos_c, src_c, ps_c, nt_c = _combine_meta(g, u, router_down)

    hs = _sc_gather(h, src_c.reshape(PAD))
    ys_c = _grouped_mm(hs, w_down, ps_c.reshape(E), nt_c.reshape(E), 2)
    out = _sc_gather(ys_c, pos_c.reshape(T))

    return out.reshape(b, s, d)


# speedup vs baseline: 1.0208x; 1.0208x over previous
import functools

import jax
import jax.numpy as jnp
from jax import lax
from jax.experimental import pallas as pl
from jax.experimental.pallas import tpu as pltpu
from jax.experimental.pallas import tpu_sc as plsc

E = 64
D_MODEL = 768
D_FF = 2048
T = 2048

TILE = 32
NT = 128
PAD = NT * TILE

_CHUNK_T = 128
_CHUNK_P = 512


def _meta_compute(ids, pos_ref, src_ref, ps_ref, nt_ref):
    e_iota = lax.broadcasted_iota(jnp.int32, (T, E), 1)
    onehot = (ids == e_iota).astype(jnp.float32)

    r_iota = lax.broadcasted_iota(jnp.int32, (_CHUNK_T, _CHUNK_T), 0)
    c_iota = lax.broadcasted_iota(jnp.int32, (_CHUNK_T, _CHUNK_T), 1)
    tri = (c_iota <= r_iota).astype(jnp.float32)
    nchunk = T // _CHUNK_T
    local = [
        jnp.dot(tri, onehot[k * _CHUNK_T:(k + 1) * _CHUNK_T, :],
                preferred_element_type=jnp.float32)
        for k in range(nchunk)
    ]
    offset = jnp.zeros((1, E), jnp.float32)
    chunks = []
    for k in range(nchunk):
        chunks.append(local[k] + offset)
        offset = offset + local[k][_CHUNK_T - 1:_CHUNK_T, :]
    csum = jnp.concatenate(chunks, axis=0)

    counts = csum[T - 1:T, :]
    tiles = jnp.floor((counts + (TILE - 1)) * (1.0 / TILE))
    e_sq_r = lax.broadcasted_iota(jnp.int32, (E, E), 0)
    e_sq_c = lax.broadcasted_iota(jnp.int32, (E, E), 1)
    stri = (e_sq_r < e_sq_c).astype(jnp.float32)
    tile_start = jnp.dot(tiles, stri, preferred_element_type=jnp.float32)
    pad_start = tile_start * float(TILE)

    rank = jnp.sum(onehot * (csum - 1.0), axis=1, keepdims=True)
    pos_f = jnp.sum(onehot * pad_start, axis=1, keepdims=True) + rank
    pos_ref[...] = pos_f.astype(jnp.int32)

    ps_ref[...] = pad_start.astype(jnp.int32)
    nt_ref[...] = tiles.astype(jnp.int32)

    pos_i = pos_f.astype(jnp.int32)
    tcol = lax.broadcasted_iota(jnp.int32, (T, 1), 0).astype(jnp.float32) + 1.0
    for r in range(PAD // _CHUNK_P):
        p_iota = lax.broadcasted_iota(jnp.int32, (T, _CHUNK_P), 1) + r * _CHUNK_P
        hit = (pos_i == p_iota).astype(jnp.float32)
        srcv = jnp.sum(hit * tcol, axis=0, keepdims=True)
        prow = (lax.broadcasted_iota(jnp.int32, (1, _CHUNK_P), 1)
                + (r * _CHUNK_P) % T).astype(jnp.float32)
        src_ref[r:r + 1, :] = jnp.where(srcv > 0.0, srcv - 1.0, prow).astype(jnp.int32)


def _route_meta2_body(x_ref, ra_ref, rb_ref,
                      pa_ref, sa_ref, psa_ref, nta_ref,
                      pb_ref, sb_ref, psb_ref, ntb_ref):
    x = x_ref[...]
    la = jnp.dot(x, ra_ref[...], preferred_element_type=jnp.float32)
    lb = jnp.dot(x, rb_ref[...], preferred_element_type=jnp.float32)
    ids_a = jnp.argmax(la, axis=1, keepdims=True).astype(jnp.int32)
    ids_b = jnp.argmax(lb, axis=1, keepdims=True).astype(jnp.int32)
    _meta_compute(ids_a, pa_ref, sa_ref, psa_ref, nta_ref)
    _meta_compute(ids_b, pb_ref, sb_ref, psb_ref, ntb_ref)


def _route_meta2(x, rwa, rwb):
    d = x.shape[1]
    meta_shapes = [
        jax.ShapeDtypeStruct((T, 1), jnp.int32),
        jax.ShapeDtypeStruct((PAD // _CHUNK_P, _CHUNK_P), jnp.int32),
        jax.ShapeDtypeStruct((1, E), jnp.int32),
        jax.ShapeDtypeStruct((1, E), jnp.int32),
    ]
    meta_specs = [
        pl.BlockSpec((T, 1), lambda: (0, 0)),
        pl.BlockSpec((PAD // _CHUNK_P, _CHUNK_P), lambda: (0, 0)),
        pl.BlockSpec((1, E), lambda: (0, 0)),
        pl.BlockSpec((1, E), lambda: (0, 0)),
    ]
    return pl.pallas_call(
        _route_meta2_body,
        out_shape=meta_shapes + meta_shapes,
        in_specs=[
            pl.BlockSpec((T, d), lambda: (0, 0)),
            pl.BlockSpec((d, E), lambda: (0, 0)),
            pl.BlockSpec((d, E), lambda: (0, 0)),
        ],
        out_specs=meta_specs + meta_specs,
    )(x, rwa, rwb)


def _gather_loop(info, n, d, table_hbm, idx_hbm, out_hbm, idx_v, rows_v, sem):
    nw = info.num_cores * info.num_subcores
    b_per_w = n // nw
    chunk = idx_v.shape[0]
    wid = lax.axis_index("s") * info.num_cores + lax.axis_index("c")
    for ci in range(b_per_w // chunk):
        base = wid * b_per_w + ci * chunk
        pltpu.sync_copy(idx_hbm.at[pl.ds(base, chunk)], idx_v)
        pltpu.async_copy(table_hbm.at[idx_v], rows_v, sem).wait()
        pltpu.sync_copy(rows_v, out_hbm.at[pl.ds(base, chunk)])


def _chunk_rows(n, d, info):
    b_per_w = n // (info.num_cores * info.num_subcores)
    budget_rows = (192 * 1024) // (d * 4)
    chunk = b_per_w
    while chunk > 8 and (chunk > budget_rows or chunk % 8 != 0):
        chunk //= 2
    return chunk


def _sc_gather(table, idx):
    n, d = idx.shape[0], table.shape[1]
    info = plsc.get_sparse_core_info()
    chunk = _chunk_rows(n, d, info)
    mesh = plsc.VectorSubcoreMesh(core_axis_name="c", subcore_axis_name="s")

    @functools.partial(
        pl.kernel,
        mesh=mesh,
        out_type=jax.ShapeDtypeStruct((n, d), jnp.float32),
        scratch_types=[
            pltpu.VMEM((chunk,), jnp.int32),
            pltpu.VMEM((chunk, d), jnp.float32),
            pltpu.SemaphoreType.DMA,
        ],
    )
    def k(table_hbm, idx_hbm, out_hbm, idx_v, rows_v, sem):
        _gather_loop(info, n, d, table_hbm, idx_hbm, out_hbm, idx_v, rows_v, sem)

    return k(table, idx)


def _sc_gather2(table_a, idx_a, table_b, idx_b):
    na, da = idx_a.shape[0], table_a.shape[1]
    nb, db = idx_b.shape[0], table_b.shape[1]
    info = plsc.get_sparse_core_info()
    ca = _chunk_rows(na, da, info)
    cb = _chunk_rows(nb, db, info)
    mesh = plsc.VectorSubcoreMesh(core_axis_name="c", subcore_axis_name="s")

    @functools.partial(
        pl.kernel,
        mesh=mesh,
        out_type=[
            jax.ShapeDtypeStruct((na, da), jnp.float32),
            jax.ShapeDtypeStruct((nb, db), jnp.float32),
        ],
        scratch_types=[
            pltpu.VMEM((ca,), jnp.int32),
            pltpu.VMEM((ca, da), jnp.float32),
            pltpu.VMEM((cb,), jnp.int32),
            pltpu.VMEM((cb, db), jnp.float32),
            pltpu.SemaphoreType.DMA,
        ],
    )
    def k(ta_hbm, ia_hbm, tb_hbm, ib_hbm, oa_hbm, ob_hbm,
          ia_v, ra_v, ib_v, rb_v, sem):
        _gather_loop(info, na, da, ta_hbm, ia_hbm, oa_hbm, ia_v, ra_v, sem)
        _gather_loop(info, nb, db, tb_hbm, ib_hbm, ob_hbm, ib_v, rb_v, sem)

    return k(table_a, idx_a, table_b, idx_b)


def _grouped_mm_body(ps_ref, nt_ref, x_ref, whi_ref, wlo_ref, o_ref):
    e = pl.program_id(1)
    start = ps_ref[e]
    whi = whi_ref[0]
    wlo = wlo_ref[0]
    hk = whi.shape[0]

    def tile_body(k, _):
        off = pl.multiple_of(start + k * TILE, TILE)
        xr = x_ref[pl.ds(off, TILE), :]
        o_ref[pl.ds(off, TILE), :] = (
            jnp.dot(xr[:, :hk], whi, preferred_element_type=jnp.float32)
            + jnp.dot(xr[:, hk:], wlo, preferred_element_type=jnp.float32))
        return _

    lax.fori_loop(0, nt_ref[e], tile_body, None)


def _grouped_mm(xs, w, ps, nt, nsplit):
    d_in, d_out = w.shape[1], w.shape[2]
    half = d_out // nsplit
    spec = pltpu.PrefetchScalarGridSpec(
        num_scalar_prefetch=2,
        grid=(nsplit, E),
        in_specs=[
            pl.BlockSpec((PAD, d_in), lambda j, e, ps, nt: (0, 0)),
            pl.BlockSpec((1, d_in // 2, half), lambda j, e, ps, nt: (e, 0, j)),
            pl.BlockSpec((1, d_in // 2, half), lambda j, e, ps, nt: (e, 1, j)),
        ],
        out_specs=pl.BlockSpec((PAD, half), lambda j, e, ps, nt: (0, j)),
    )
    return pl.pallas_call(
        _grouped_mm_body,
        grid_spec=spec,
        out_shape=jax.ShapeDtypeStruct((PAD, d_out), jnp.float32),
    )(ps, nt, xs, w, w)


_CBLK = 256


def _combine_meta_body(g_ref, u_ref, rd_ref, h_ref, pc_ref, sc_ref,
                       psc_ref, ntc_ref, ids_acc):
    i = pl.program_id(0)
    g = g_ref[...]
    h = g * lax.logistic(g) * u_ref[...]
    h_ref[...] = h
    lc = jnp.dot(h, rd_ref[...], preferred_element_type=jnp.float32)
    ids_acc[pl.ds(i * _CBLK, _CBLK), :] = (
        jnp.argmax(lc, axis=1, keepdims=True).astype(jnp.int32))

    @pl.when(i == T // _CBLK - 1)
    def _finish():
        _meta_compute(ids_acc[...], pc_ref, sc_ref, psc_ref, ntc_ref)


def _combine_meta(g, u, rdown):
    return pl.pallas_call(
        _combine_meta_body,
        grid=(T // _CBLK,),
        out_shape=[
            jax.ShapeDtypeStruct((T, D_FF), jnp.float32),
            jax.ShapeDtypeStruct((T, 1), jnp.int32),
            jax.ShapeDtypeStruct((PAD // _CHUNK_P, _CHUNK_P), jnp.int32),
            jax.ShapeDtypeStruct((1, E), jnp.int32),
            jax.ShapeDtypeStruct((1, E), jnp.int32),
        ],
        in_specs=[
            pl.BlockSpec((_CBLK, D_FF), lambda i: (i, 0)),
            pl.BlockSpec((_CBLK, D_FF), lambda i: (i, 0)),
            pl.BlockSpec((D_FF, E), lambda i: (0, 0)),
        ],
        out_specs=[
            pl.BlockSpec((_CBLK, D_FF), lambda i: (i, 0)),
            pl.BlockSpec((T, 1), lambda i: (0, 0)),
            pl.BlockSpec((PAD // _CHUNK_P, _CHUNK_P), lambda i: (0, 0)),
            pl.BlockSpec((1, E), lambda i: (0, 0)),
            pl.BlockSpec((1, E), lambda i: (0, 0)),
        ],
        scratch_shapes=[pltpu.VMEM((T, 1), jnp.int32)],
    )(g, u, rdown)


def kernel(hidden_state, router_gate, w_gate, router_up, w_up, router_down, w_down):
    b, s, d = hidden_state.shape
    x = hidden_state.reshape(b * s, d)

    (pos_a, src_a, ps_a, nt_a,
     pos_b, src_b, ps_b, nt_b) = _route_meta2(x, router_gate, router_up)

    xs_a = _sc_gather(x, src_a.reshape(PAD))
    ys_a = _grouped_mm(xs_a, w_gate, ps_a.reshape(E), nt_a.reshape(E), 2)
    xs_b = _sc_gather(x, src_b.reshape(PAD))
    g = _sc_gather(ys_a, pos_a.reshape(T))
    ys_b = _grouped_mm(xs_b, w_up, ps_b.reshape(E), nt_b.reshape(E), 2)
    u = _sc_gather(ys_b, pos_b.reshape(T))

    h, pos_c, src_c, ps_c, nt_c = _combine_meta(g, u, router_down)

    hs = _sc_gather(h, src_c.reshape(PAD))
    ys_c = _grouped_mm(hs, w_down, ps_c.reshape(E), nt_c.reshape(E), 2)
    out = _sc_gather(ys_c, pos_c.reshape(T))

    return out.reshape(b, s, d)


# speedup vs baseline: 1.0227x; 1.0018x over previous
import functools

import jax
import jax.numpy as jnp
from jax import lax
from jax.experimental import pallas as pl
from jax.experimental.pallas import tpu as pltpu
from jax.experimental.pallas import tpu_sc as plsc

E = 64
D_MODEL = 768
D_FF = 2048
T = 2048

TILE = 32
NT = 128
PAD = NT * TILE

_CHUNK_T = 128
_CHUNK_P = 512


def _meta_compute(ids, pos_ref, src_ref, ps_ref, nt_ref):
    e_iota = lax.broadcasted_iota(jnp.int32, (T, E), 1)
    onehot = (ids == e_iota).astype(jnp.float32)

    r_iota = lax.broadcasted_iota(jnp.int32, (_CHUNK_T, _CHUNK_T), 0)
    c_iota = lax.broadcasted_iota(jnp.int32, (_CHUNK_T, _CHUNK_T), 1)
    tri = (c_iota <= r_iota).astype(jnp.float32)
    nchunk = T // _CHUNK_T
    local = [
        jnp.dot(tri, onehot[k * _CHUNK_T:(k + 1) * _CHUNK_T, :],
                preferred_element_type=jnp.float32)
        for k in range(nchunk)
    ]
    offset = jnp.zeros((1, E), jnp.float32)
    chunks = []
    for k in range(nchunk):
        chunks.append(local[k] + offset)
        offset = offset + local[k][_CHUNK_T - 1:_CHUNK_T, :]
    csum = jnp.concatenate(chunks, axis=0)

    counts = csum[T - 1:T, :]
    tiles = jnp.floor((counts + (TILE - 1)) * (1.0 / TILE))
    e_sq_r = lax.broadcasted_iota(jnp.int32, (E, E), 0)
    e_sq_c = lax.broadcasted_iota(jnp.int32, (E, E), 1)
    stri = (e_sq_r < e_sq_c).astype(jnp.float32)
    tile_start = jnp.dot(tiles, stri, preferred_element_type=jnp.float32)
    pad_start = tile_start * float(TILE)

    rank = jnp.sum(onehot * (csum - 1.0), axis=1, keepdims=True)
    pos_f = jnp.sum(onehot * pad_start, axis=1, keepdims=True) + rank
    pos_ref[...] = pos_f.astype(jnp.int32)

    ps_ref[...] = pad_start.astype(jnp.int32)
    nt_ref[...] = tiles.astype(jnp.int32)

    pos_i = pos_f.astype(jnp.int32)
    tcol = lax.broadcasted_iota(jnp.int32, (T, 1), 0).astype(jnp.float32) + 1.0
    for r in range(PAD // _CHUNK_P):
        p_iota = lax.broadcasted_iota(jnp.int32, (T, _CHUNK_P), 1) + r * _CHUNK_P
        hit = (pos_i == p_iota).astype(jnp.float32)
        srcv = jnp.sum(hit * tcol, axis=0, keepdims=True)
        prow = (lax.broadcasted_iota(jnp.int32, (1, _CHUNK_P), 1)
                + (r * _CHUNK_P) % T).astype(jnp.float32)
        src_ref[r:r + 1, :] = jnp.where(srcv > 0.0, srcv - 1.0, prow).astype(jnp.int32)


def _route_meta2_body(x_ref, ra_ref, rb_ref,
                      pa_ref, sa_ref, psa_ref, nta_ref,
                      pb_ref, sb_ref, psb_ref, ntb_ref):
    x = x_ref[...]
    la = jnp.dot(x, ra_ref[...], preferred_element_type=jnp.float32)
    lb = jnp.dot(x, rb_ref[...], preferred_element_type=jnp.float32)
    ids_a = jnp.argmax(la, axis=1, keepdims=True).astype(jnp.int32)
    ids_b = jnp.argmax(lb, axis=1, keepdims=True).astype(jnp.int32)
    _meta_compute(ids_a, pa_ref, sa_ref, psa_ref, nta_ref)
    _meta_compute(ids_b, pb_ref, sb_ref, psb_ref, ntb_ref)


def _route_meta2(x, rwa, rwb):
    d = x.shape[1]
    meta_shapes = [
        jax.ShapeDtypeStruct((T, 1), jnp.int32),
        jax.ShapeDtypeStruct((PAD // _CHUNK_P, _CHUNK_P), jnp.int32),
        jax.ShapeDtypeStruct((1, E), jnp.int32),
        jax.ShapeDtypeStruct((1, E), jnp.int32),
    ]
    meta_specs = [
        pl.BlockSpec((T, 1), lambda: (0, 0)),
        pl.BlockSpec((PAD // _CHUNK_P, _CHUNK_P), lambda: (0, 0)),
        pl.BlockSpec((1, E), lambda: (0, 0)),
        pl.BlockSpec((1, E), lambda: (0, 0)),
    ]
    return pl.pallas_call(
        _route_meta2_body,
        out_shape=meta_shapes + meta_shapes,
        in_specs=[
            pl.BlockSpec((T, d), lambda: (0, 0)),
            pl.BlockSpec((d, E), lambda: (0, 0)),
            pl.BlockSpec((d, E), lambda: (0, 0)),
        ],
        out_specs=meta_specs + meta_specs,
    )(x, rwa, rwb)


def _gather_loop(info, n, d, table_hbm, idx_hbm, out_hbm, idx_v, rows_v, sem):
    nw = info.num_cores * info.num_subcores
    b_per_w = n // nw
    chunk = idx_v.shape[0]
    wid = lax.axis_index("s") * info.num_cores + lax.axis_index("c")
    for ci in range(b_per_w // chunk):
        base = wid * b_per_w + ci * chunk
        pltpu.sync_copy(idx_hbm.at[pl.ds(base, chunk)], idx_v)
        pltpu.async_copy(table_hbm.at[idx_v], rows_v, sem).wait()
        pltpu.sync_copy(rows_v, out_hbm.at[pl.ds(base, chunk)])


def _chunk_rows(n, d, info):
    b_per_w = n // (info.num_cores * info.num_subcores)
    budget_rows = (192 * 1024) // (d * 4)
    chunk = b_per_w
    while chunk > 8 and (chunk > budget_rows or chunk % 8 != 0):
        chunk //= 2
    return chunk


def _sc_gather(table, idx):
    n, d = idx.shape[0], table.shape[1]
    info = plsc.get_sparse_core_info()
    chunk = _chunk_rows(n, d, info)
    mesh = plsc.VectorSubcoreMesh(core_axis_name="c", subcore_axis_name="s")

    @functools.partial(
        pl.kernel,
        mesh=mesh,
        out_type=jax.ShapeDtypeStruct((n, d), jnp.float32),
        scratch_types=[
            pltpu.VMEM((chunk,), jnp.int32),
            pltpu.VMEM((chunk, d), jnp.float32),
            pltpu.SemaphoreType.DMA,
        ],
    )
    def k(table_hbm, idx_hbm, out_hbm, idx_v, rows_v, sem):
        _gather_loop(info, n, d, table_hbm, idx_hbm, out_hbm, idx_v, rows_v, sem)

    return k(table, idx)


def _grouped_mm_body(ps_ref, nt_ref, x_ref, whi_ref, wlo_ref, o_ref):
    e = pl.program_id(1)
    start = ps_ref[e]
    whi = whi_ref[0]
    wlo = wlo_ref[0]
    hk = whi.shape[0]

    def tile_body(k, _):
        off = pl.multiple_of(start + k * TILE, TILE)
        xr = x_ref[pl.ds(off, TILE), :]
        o_ref[pl.ds(off, TILE), :] = (
            jnp.dot(xr[:, :hk], whi, preferred_element_type=jnp.float32)
            + jnp.dot(xr[:, hk:], wlo, preferred_element_type=jnp.float32))
        return _

    lax.fori_loop(0, nt_ref[e], tile_body, None)


def _grouped_mm(xs, w, ps, nt, nsplit):
    d_in, d_out = w.shape[1], w.shape[2]
    half = d_out // nsplit
    spec = pltpu.PrefetchScalarGridSpec(
        num_scalar_prefetch=2,
        grid=(nsplit, E),
        in_specs=[
            pl.BlockSpec((PAD, d_in), lambda j, e, ps, nt: (0, 0)),
            pl.BlockSpec((1, d_in // 2, half), lambda j, e, ps, nt: (e, 0, j)),
            pl.BlockSpec((1, d_in // 2, half), lambda j, e, ps, nt: (e, 1, j)),
        ],
        out_specs=pl.BlockSpec((PAD, half), lambda j, e, ps, nt: (0, j)),
    )
    return pl.pallas_call(
        _grouped_mm_body,
        grid_spec=spec,
        out_shape=jax.ShapeDtypeStruct((PAD, d_out), jnp.float32),
    )(ps, nt, xs, w, w)


_CBLK = 256


def _combine_meta_body(g_ref, u_ref, rd_ref, h_ref, pc_ref, sc_ref,
                       psc_ref, ntc_ref, ids_acc):
    i = pl.program_id(0)
    g = g_ref[...]
    h = g * lax.logistic(g) * u_ref[...]
    h_ref[...] = h
    lc = jnp.dot(h, rd_ref[...], preferred_element_type=jnp.float32)
    ids_acc[pl.ds(i * _CBLK, _CBLK), :] = (
        jnp.argmax(lc, axis=1, keepdims=True).astype(jnp.int32))

    @pl.when(i == T // _CBLK - 1)
    def _finish():
        _meta_compute(ids_acc[...], pc_ref, sc_ref, psc_ref, ntc_ref)


def _combine_meta(g, u, rdown):
    return pl.pallas_call(
        _combine_meta_body,
        grid=(T // _CBLK,),
        out_shape=[
            jax.ShapeDtypeStruct((T, D_FF), jnp.float32),
            jax.ShapeDtypeStruct((T, 1), jnp.int32),
            jax.ShapeDtypeStruct((PAD // _CHUNK_P, _CHUNK_P), jnp.int32),
            jax.ShapeDtypeStruct((1, E), jnp.int32),
            jax.ShapeDtypeStruct((1, E), jnp.int32),
        ],
        in_specs=[
            pl.BlockSpec((_CBLK, D_FF), lambda i: (i, 0)),
            pl.BlockSpec((_CBLK, D_FF), lambda i: (i, 0)),
            pl.BlockSpec((D_FF, E), lambda i: (0, 0)),
        ],
        out_specs=[
            pl.BlockSpec((_CBLK, D_FF), lambda i: (i, 0)),
            pl.BlockSpec((T, 1), lambda i: (0, 0)),
            pl.BlockSpec((PAD // _CHUNK_P, _CHUNK_P), lambda i: (0, 0)),
            pl.BlockSpec((1, E), lambda i: (0, 0)),
            pl.BlockSpec((1, E), lambda i: (0, 0)),
        ],
        scratch_shapes=[pltpu.VMEM((T, 1), jnp.int32)],
    )(g, u, rdown)


def kernel(hidden_state, router_gate, w_gate, router_up, w_up, router_down, w_down):
    b, s, d = hidden_state.shape
    x = hidden_state.reshape(b * s, d)

    (pos_a, src_a, ps_a, nt_a,
     pos_b, src_b, ps_b, nt_b) = _route_meta2(x, router_gate, router_up)

    xs_a = _sc_gather(x, src_a.reshape(PAD))
    ys_a = _grouped_mm(xs_a, w_gate, ps_a.reshape(E), nt_a.reshape(E), 2)
    xs_b = _sc_gather(x, src_b.reshape(PAD))
    g = _sc_gather(ys_a, pos_a.reshape(T))
    ys_b = _grouped_mm(xs_b, w_up, ps_b.reshape(E), nt_b.reshape(E), 2)
    u = _sc_gather(ys_b, pos_b.reshape(T))

    h, pos_c, src_c, ps_c, nt_c = _combine_meta(g, u, router_down)

    hs = _sc_gather(h, src_c.reshape(PAD))
    ys_c = _grouped_mm(hs, w_down, ps_c.reshape(E), nt_c.reshape(E), 2)
    out = _sc_gather(ys_c, pos_c.reshape(T))

    return out.reshape(b, s, d)
